# baseline (speedup 1.0000x reference)
import jax
import jax.numpy as jnp
from jax import lax
from jax.experimental import pallas as pl
from jax.experimental.pallas import tpu as pltpu
from jax.experimental.pallas import tpu_sc as plsc

_DIM = 64
_BATCH = 16384
_NNEG = 20
_NC = 2
_NS = 16
_NW = _NC * _NS
_U = 128
_WU = _BATCH // _NW // _U
_GU = _BATCH * _NNEG // _NW // _U
_TU = 2 * _WU + _GU


def _body(widx, pidx, nidx, w_in, w_out, o_word, o_pos, o_neg,
          idx_v, rows_v, sem):
    wid = lax.axis_index("s") * _NC + lax.axis_index("c")

    pltpu.sync_copy(widx.at[pl.ds(wid * _WU, _WU)], idx_v.at[pl.ds(0, _WU)])
    pltpu.sync_copy(pidx.at[pl.ds(wid * _WU, _WU)], idx_v.at[pl.ds(_WU, _WU)])
    pltpu.sync_copy(nidx.at[pl.ds(wid * _GU, _GU)], idx_v.at[pl.ds(2 * _WU, _GU)])

    for j in range(_WU):
        pltpu.async_copy(w_in.at[idx_v.at[j]], rows_v, sem).wait()
        pltpu.sync_copy(rows_v, o_word.at[pl.ds((wid * _WU + j) * _U, _U)])

    for j in range(_WU):
        pltpu.async_copy(w_out.at[idx_v.at[_WU + j]], rows_v, sem).wait()
        pltpu.sync_copy(rows_v, o_pos.at[pl.ds((wid * _WU + j) * _U, _U)])

    def neg_unit(j, carry):
        pltpu.async_copy(w_out.at[idx_v.at[2 * _WU + j]], rows_v, sem).wait()
        pltpu.sync_copy(rows_v, o_neg.at[pl.ds((wid * _GU + j) * _U, _U)])
        return carry

    lax.fori_loop(0, _GU, neg_unit, 0)


def kernel(word, positive, negatives, w_input, w_output):
    widx = word.reshape(_BATCH // _U, _U).astype(jnp.int32)
    pidx = positive.reshape(_BATCH // _U, _U).astype(jnp.int32)
    nidx = negatives.reshape(_BATCH * _NNEG // _U, _U).astype(jnp.int32)
    f = pl.kernel(
        _body,
        out_type=(
            jax.ShapeDtypeStruct((_BATCH, _DIM), jnp.float32),
            jax.ShapeDtypeStruct((_BATCH, _DIM), jnp.float32),
            jax.ShapeDtypeStruct((_BATCH * _NNEG, _DIM), jnp.float32),
        ),
        mesh=plsc.VectorSubcoreMesh(core_axis_name="c", subcore_axis_name="s"),
        scratch_types=[
            pltpu.VMEM((_TU, _U), jnp.int32),
            pltpu.VMEM((_U, _DIM), jnp.float32),
            pltpu.SemaphoreType.DMA,
        ],
        compiler_params=pltpu.CompilerParams(use_tc_tiling_on_sc=False),
    )
    o_word, o_pos, o_neg = f(widx, pidx, nidx, w_input, w_output)
    return o_word, o_pos, o_neg.reshape(_BATCH, _NNEG, _DIM)


# speedup vs baseline: 1.0438x; 1.0438x over previous
import jax
import jax.numpy as jnp
from jax import lax
from jax.experimental import pallas as pl
from jax.experimental.pallas import tpu as pltpu
from jax.experimental.pallas import tpu_sc as plsc

_DIM = 64
_BATCH = 16384
_NNEG = 20
_NC = 2
_NS = 16
_NW = _NC * _NS
_U = 128
_WU = _BATCH // _NW // _U
_GU = _BATCH * _NNEG // _NW // _U
_TU = 2 * _WU + _GU
_R = 8


def _body(widx, pidx, nidx, w_in, w_out, o_word, o_pos, o_neg,
          idx_v, bufs, gsem, ssem):
    wid = lax.axis_index("s") * _NC + lax.axis_index("c")

    pltpu.sync_copy(widx.at[pl.ds(wid * _WU, _WU)], idx_v.at[pl.ds(0, _WU)])
    pltpu.sync_copy(pidx.at[pl.ds(wid * _WU, _WU)], idx_v.at[pl.ds(_WU, _WU)])
    pltpu.sync_copy(nidx.at[pl.ds(wid * _GU, _GU)], idx_v.at[pl.ds(2 * _WU, _GU)])

    def table(u):
        return w_in if u < _WU else w_out

    def dst(u):
        if u < _WU:
            return o_word.at[pl.ds((wid * _WU + u) * _U, _U)]
        if u < 2 * _WU:
            return o_pos.at[pl.ds((wid * _WU + u - _WU) * _U, _U)]
        return o_neg.at[pl.ds((wid * _GU + u - 2 * _WU) * _U, _U)]

    def gf(u):
        pltpu.async_copy(table(u).at[idx_v.at[u]], bufs.at[u % _R],
                         gsem.at[u % _R])

    def gw(u):
        pltpu.make_async_copy(table(u).at[idx_v.at[u]], bufs.at[u % _R],
                              gsem.at[u % _R]).wait()

    def sf(u):
        pltpu.async_copy(bufs.at[u % _R], dst(u), ssem.at[u % _R])

    def sw_slot(s):
        pltpu.make_async_copy(bufs.at[s], o_neg.at[pl.ds(0, _U)],
                              ssem.at[s]).wait()

    for u in range(12):
        if u >= 4:
            gw(u - 4)
            sf(u - 4)
        if u >= 8:
            sw_slot(u - 8)
        gf(u)

    def group(k, carry):
        for j in range(_R):
            u = 12 + k * _R + j
            s2 = (j + 4) % _R
            pltpu.make_async_copy(w_out.at[idx_v.at[u - 4]], bufs.at[j],
                                  gsem.at[j]).wait()
            pltpu.async_copy(
                bufs.at[j],
                o_neg.at[pl.ds((wid * _GU + k * _R + j) * _U, _U)],
                ssem.at[j])
            sw_slot(s2)
            pltpu.async_copy(w_out.at[idx_v.at[u]], bufs.at[s2],
                             gsem.at[s2])
        return carry

    lax.fori_loop(0, (_TU - 24) // _R, group, 0)

    for u in range(_TU - 12, _TU + _R):
        if u - 4 < _TU:
            gw(u - 4)
            sf(u - 4)
        if u - 8 < _TU:
            sw_slot((u - 8) % _R)
        if u < _TU:
            gf(u)


def kernel(word, positive, negatives, w_input, w_output):
    widx = word.reshape(_BATCH // _U, _U).astype(jnp.int32)
    pidx = positive.reshape(_BATCH // _U, _U).astype(jnp.int32)
    nidx = negatives.reshape(_BATCH * _NNEG // _U, _U).astype(jnp.int32)
    f = pl.kernel(
        _body,
        out_type=(
            jax.ShapeDtypeStruct((_BATCH, _DIM), jnp.float32),
            jax.ShapeDtypeStruct((_BATCH, _DIM), jnp.float32),
            jax.ShapeDtypeStruct((_BATCH * _NNEG, _DIM), jnp.float32),
        ),
        mesh=plsc.VectorSubcoreMesh(core_axis_name="c", subcore_axis_name="s"),
        scratch_types=[
            pltpu.VMEM((_TU, _U), jnp.int32),
            pltpu.VMEM((_R, _U, _DIM), jnp.float32),
            pltpu.SemaphoreType.DMA((_R,)),
            pltpu.SemaphoreType.DMA((_R,)),
        ],
        compiler_params=pltpu.CompilerParams(use_tc_tiling_on_sc=False),
    )
    o_word, o_pos, o_neg = f(widx, pidx, nidx, w_input, w_output)
    return o_word, o_pos, o_neg.reshape(_BATCH, _NNEG, _DIM)


# speedup vs baseline: 1.0458x; 1.0019x over previous
import jax
import jax.numpy as jnp
from jax import lax
from jax.experimental import pallas as pl
from jax.experimental.pallas import tpu as pltpu
from jax.experimental.pallas import tpu_sc as plsc

_DIM = 64
_BATCH = 16384
_NNEG = 20
_NC = 2
_NS = 16
_NW = _NC * _NS
_U = 128
_WU = _BATCH // _NW // _U
_GU = _BATCH * _NNEG // _NW // _U
_TU = 2 * _WU + _GU
_R = 8


def _body(widx, pidx, nidx, w_in, w_out, o_word, o_pos, o_neg,
          idx_v, bufs, gsem, ssem):
    wid = lax.axis_index("s") * _NC + lax.axis_index("c")

    pltpu.sync_copy(widx.at[pl.ds(wid * _WU, _WU)], idx_v.at[pl.ds(0, _WU)])
    pltpu.sync_copy(pidx.at[pl.ds(wid * _WU, _WU)], idx_v.at[pl.ds(_WU, _WU)])
    pltpu.sync_copy(nidx.at[pl.ds(wid * _GU, _GU)], idx_v.at[pl.ds(2 * _WU, _GU)])

    def table(u):
        return w_in if u < _WU else w_out

    def dst(u):
        if u < _WU:
            return o_word.at[pl.ds((wid * _WU + u) * _U, _U)]
        if u < 2 * _WU:
            return o_pos.at[pl.ds((wid * _WU + u - _WU) * _U, _U)]
        return o_neg.at[pl.ds((wid * _GU + u - 2 * _WU) * _U, _U)]

    def gf(u):
        pltpu.async_copy(table(u).at[idx_v.at[u]], bufs.at[u % _R],
                         gsem.at[u % _R])

    def gw(u):
        pltpu.make_async_copy(table(u).at[idx_v.at[u]], bufs.at[u % _R],
                              gsem.at[u % _R]).wait()

    def sf(u):
        pltpu.async_copy(bufs.at[u % _R], dst(u), ssem.at[u % _R])

    def sw_slot(s):
        pltpu.make_async_copy(bufs.at[s], o_neg.at[pl.ds(0, _U)],
                              ssem.at[s]).wait()

    for u in range(12):
        if u >= 4:
            gw(u - 4)
            sf(u - 4)
        if u >= 8:
            sw_slot(u - 8)
        gf(u)

    def group(k, carry):
        for j in range(_R):
            u = 12 + k * _R + j
            s2 = (j + 4) % _R
            pltpu.make_async_copy(w_out.at[idx_v.at[u - 4]], bufs.at[j],
                                  gsem.at[j]).wait()
            pltpu.async_copy(
                bufs.at[j],
                o_neg.at[pl.ds((wid * _GU + k * _R + j) * _U, _U)],
                ssem.at[j])
            sw_slot(s2)
            pltpu.async_copy(w_out.at[idx_v.at[u]], bufs.at[s2],
                             gsem.at[s2])
        return carry

    lax.fori_loop(0, (_TU - 24) // _R, group, 0)

    for u in range(_TU - 12, _TU + _R):
        if u - 4 < _TU:
            gw(u - 4)
            sf(u - 4)
        if u - 8 < _TU:
            sw_slot((u - 8) % _R)
        if u < _TU:
            gf(u)


def kernel(word, positive, negatives, w_input, w_output):
    widx = word.reshape(_BATCH // _U, _U).astype(jnp.int32)
    pidx = positive.reshape(_BATCH // _U, _U).astype(jnp.int32)
    nidx = negatives.reshape(_BATCH * _NNEG // _U, _U).astype(jnp.int32)
    f = pl.kernel(
        _body,
        out_type=(
            jax.ShapeDtypeStruct((_BATCH, _DIM), jnp.float32),
            jax.ShapeDtypeStruct((_BATCH, _DIM), jnp.float32),
            jax.ShapeDtypeStruct((_BATCH * _NNEG, _DIM), jnp.float32),
        ),
        mesh=plsc.VectorSubcoreMesh(core_axis_name="c", subcore_axis_name="s"),
        scratch_types=[
            pltpu.VMEM((_TU, _U), jnp.int32),
            pltpu.VMEM((_R, _U, _DIM), jnp.float32),
            pltpu.SemaphoreType.DMA((_R,)),
            pltpu.SemaphoreType.DMA((_R,)),
        ],
        compiler_params=pltpu.CompilerParams(use_tc_tiling_on_sc=False),
        cost_estimate=pl.CostEstimate(
            flops=0, bytes_accessed=190_000_000, transcendentals=0),
    )
    o_word, o_pos, o_neg = f(widx, pidx, nidx, w_input, w_output)
    return o_word, o_pos, o_neg.reshape(_BATCH, _NNEG, _DIM)
